# baseline (speedup 1.0000x reference)
import jax
import jax.numpy as jnp
from jax import lax
from jax.experimental import pallas as pl
from jax.experimental.pallas import tpu as pltpu
from jax.experimental.pallas import tpu_sc as plsc

_NC = 2
_NS = 16
_NW = _NC * _NS
_LANES = 128
_PAD_BLK = 8192
_NB = 8
_EPI_BLK = 32


def _widen_table(emb_weight):
    vocab1, emb_dim = emb_weight.shape
    grid = (vocab1 + _PAD_BLK - 1) // _PAD_BLK

    def pad_body(src_ref, dst_ref):
        dst_ref[...] = jnp.pad(
            src_ref[...], ((0, 0), (0, _LANES - emb_dim)))

    return pl.pallas_call(
        pad_body,
        grid=(grid,),
        in_specs=[pl.BlockSpec((_PAD_BLK, emb_dim), lambda i: (i, 0))],
        out_specs=pl.BlockSpec((_PAD_BLK, _LANES), lambda i: (i, 0)),
        out_shape=jax.ShapeDtypeStruct((vocab1, _LANES), jnp.float32),
    )(emb_weight)


def _narrow_out(wide3, batch, hist, emb_dim):
    def epi_body(src_ref, dst_ref):
        dst_ref[...] = src_ref[:, :, :emb_dim]

    return pl.pallas_call(
        epi_body,
        grid=(batch // _EPI_BLK,),
        in_specs=[pl.BlockSpec((_EPI_BLK, hist, _LANES), lambda i: (i, 0, 0))],
        out_specs=pl.BlockSpec((_EPI_BLK, hist, emb_dim),
                               lambda i: (i, 0, 0)),
        out_shape=jax.ShapeDtypeStruct((batch, hist, emb_dim), jnp.float32),
    )(wide3)


def kernel(x, emb_weight):
    batch, hist = x.shape
    vocab1, emb_dim = emb_weight.shape
    batches_per_worker = batch // _NW
    groups_per_worker = batches_per_worker // _NB

    tbl_wide = jnp.pad(emb_weight, ((0, 0), (0, _LANES - emb_dim)))

    mesh = plsc.VectorSubcoreMesh(core_axis_name="c", subcore_axis_name="s")

    @pl.kernel(
        out_type=jax.ShapeDtypeStruct((batch, hist, _LANES), jnp.float32),
        mesh=mesh,
        scratch_types=[
            pltpu.VMEM((_NB, hist), jnp.int32),
            pltpu.VMEM((_NB, hist, _LANES), jnp.float32),
            pltpu.SemaphoreType.DMA,
        ],
    )
    def gather_kernel(tbl_hbm, idx_hbm, out_hbm, idx_v, rows_v, sem):
        wid = lax.axis_index("s") * _NC + lax.axis_index("c")
        b0 = wid * batches_per_worker

        @pl.loop(0, groups_per_worker)
        def _(g):
            b = b0 + g * _NB
            pltpu.sync_copy(idx_hbm.at[pl.ds(b, _NB)], idx_v)
            copies = [
                pltpu.async_copy(
                    tbl_hbm.at[idx_v.at[j]],
                    rows_v.at[j],
                    sem,
                )
                for j in range(_NB)
            ]
            for c in copies:
                c.wait()
            pltpu.sync_copy(rows_v, out_hbm.at[pl.ds(b, _NB)])

    wide3 = gather_kernel(tbl_wide, x)
    return wide3[:, :, :emb_dim]


# speedup vs baseline: 1.0513x; 1.0513x over previous
import jax
import jax.numpy as jnp
from jax import lax
from jax.experimental import pallas as pl
from jax.experimental.pallas import tpu as pltpu
from jax.experimental.pallas import tpu_sc as plsc

_NC = 2
_NS = 16
_NW = _NC * _NS
_LANES = 128
_PAD_BLK = 8192
_NB = 16
_EPI_BLK = 32


def _widen_table(emb_weight):
    vocab1, emb_dim = emb_weight.shape
    grid = (vocab1 + _PAD_BLK - 1) // _PAD_BLK

    def pad_body(src_ref, dst_ref):
        dst_ref[...] = jnp.pad(
            src_ref[...], ((0, 0), (0, _LANES - emb_dim)))

    return pl.pallas_call(
        pad_body,
        grid=(grid,),
        in_specs=[pl.BlockSpec((_PAD_BLK, emb_dim), lambda i: (i, 0))],
        out_specs=pl.BlockSpec((_PAD_BLK, _LANES), lambda i: (i, 0)),
        out_shape=jax.ShapeDtypeStruct((vocab1, _LANES), jnp.float32),
    )(emb_weight)


def _narrow_out(wide3, batch, hist, emb_dim):
    def epi_body(src_ref, dst_ref):
        dst_ref[...] = src_ref[:, :, :emb_dim]

    return pl.pallas_call(
        epi_body,
        grid=(batch // _EPI_BLK,),
        in_specs=[pl.BlockSpec((_EPI_BLK, hist, _LANES), lambda i: (i, 0, 0))],
        out_specs=pl.BlockSpec((_EPI_BLK, hist, emb_dim),
                               lambda i: (i, 0, 0)),
        out_shape=jax.ShapeDtypeStruct((batch, hist, emb_dim), jnp.float32),
    )(wide3)


def kernel(x, emb_weight):
    batch, hist = x.shape
    vocab1, emb_dim = emb_weight.shape
    batches_per_worker = batch // _NW
    groups_per_worker = batches_per_worker // _NB

    tbl_wide = jnp.pad(emb_weight, ((0, 0), (0, _LANES - emb_dim)))

    mesh = plsc.VectorSubcoreMesh(core_axis_name="c", subcore_axis_name="s")

    @pl.kernel(
        out_type=jax.ShapeDtypeStruct((batch, hist, _LANES), jnp.float32),
        mesh=mesh,
        scratch_types=[
            pltpu.VMEM((_NB, hist), jnp.int32),
            pltpu.VMEM((_NB, hist, _LANES), jnp.float32),
            pltpu.SemaphoreType.DMA,
        ],
    )
    def gather_kernel(tbl_hbm, idx_hbm, out_hbm, idx_v, rows_v, sem):
        wid = lax.axis_index("s") * _NC + lax.axis_index("c")
        b0 = wid * batches_per_worker

        @pl.loop(0, groups_per_worker)
        def _(g):
            b = b0 + g * _NB
            pltpu.sync_copy(idx_hbm.at[pl.ds(b, _NB)], idx_v)
            copies = [
                pltpu.async_copy(
                    tbl_hbm.at[idx_v.at[j]],
                    rows_v.at[j],
                    sem,
                )
                for j in range(_NB)
            ]
            for c in copies:
                c.wait()
            pltpu.sync_copy(rows_v, out_hbm.at[pl.ds(b, _NB)])

    wide3 = gather_kernel(tbl_wide, x)
    return wide3[:, :, :emb_dim]


# speedup vs baseline: 1.0569x; 1.0053x over previous
import jax
import jax.numpy as jnp
from jax import lax
from jax.experimental import pallas as pl
from jax.experimental.pallas import tpu as pltpu
from jax.experimental.pallas import tpu_sc as plsc

_NC = 2
_NS = 16
_NW = _NC * _NS
_LANES = 128
_NB = 8


def kernel(x, emb_weight):
    batch, hist = x.shape
    vocab1, emb_dim = emb_weight.shape
    batches_per_worker = batch // _NW
    groups_per_worker = batches_per_worker // _NB
    pairs = groups_per_worker // 2

    tbl_wide = jnp.pad(emb_weight, ((0, 0), (0, _LANES - emb_dim)))

    mesh = plsc.VectorSubcoreMesh(core_axis_name="c", subcore_axis_name="s")

    @pl.kernel(
        out_type=jax.ShapeDtypeStruct((batch, hist, _LANES), jnp.float32),
        mesh=mesh,
        scratch_types=[
            pltpu.VMEM((_NB, hist), jnp.int32),
            pltpu.VMEM((_NB, hist), jnp.int32),
            pltpu.VMEM((_NB, hist, _LANES), jnp.float32),
            pltpu.VMEM((_NB, hist, _LANES), jnp.float32),
            pltpu.SemaphoreType.DMA,
            pltpu.SemaphoreType.DMA,
        ],
    )
    def gather_kernel(tbl_hbm, idx_hbm, out_hbm,
                      idx0, idx1, rows0, rows1, gsem, wsem):
        wid = lax.axis_index("s") * _NC + lax.axis_index("c")
        b0 = wid * batches_per_worker

        def run_group(g, idx_v, rows_v):
            b = b0 + g * _NB
            @pl.when(g >= 2)
            def _():
                pltpu.make_async_copy(
                    rows_v, out_hbm.at[pl.ds(b0, _NB)], wsem).wait()

            pltpu.sync_copy(idx_hbm.at[pl.ds(b, _NB)], idx_v)
            copies = [
                pltpu.async_copy(
                    tbl_hbm.at[idx_v.at[j]], rows_v.at[j], gsem)
                for j in range(_NB)
            ]
            for c in copies:
                c.wait()
            pltpu.async_copy(rows_v, out_hbm.at[pl.ds(b, _NB)], wsem)

        @pl.loop(0, pairs)
        def _(p):
            run_group(2 * p, idx0, rows0)
            run_group(2 * p + 1, idx1, rows1)

        pltpu.make_async_copy(rows0, out_hbm.at[pl.ds(b0, _NB)], wsem).wait()
        pltpu.make_async_copy(rows1, out_hbm.at[pl.ds(b0, _NB)], wsem).wait()

    wide3 = gather_kernel(tbl_wide, x)
    return wide3[:, :, :emb_dim]
